# Initial kernel scaffold; baseline (speedup 1.0000x reference)
import jax
import jax.numpy as jnp
from jax.experimental import pallas as pl


def kernel(d_model, max_len, tok_table, pos_table):
    raise NotImplementedError("write your pallas kernel here")


# speedup vs baseline: 1.6506x; 1.6506x over previous
import functools

import jax
import jax.numpy as jnp
from jax import lax
from jax.experimental import pallas as pl
from jax.experimental.pallas import tpu as pltpu
from jax.experimental.pallas import tpu_sc as plsc

_INFO = plsc.get_sparse_core_info()
_NC, _NS, _L = _INFO.num_cores, _INFO.num_subcores, _INFO.num_lanes
_NW = _NC * _NS

_CHUNK = 128


def _build_lookup(n_total, n_chunks, d):
    b_per_w = n_chunks * _CHUNK
    mesh = plsc.VectorSubcoreMesh(core_axis_name="c", subcore_axis_name="s")

    @functools.partial(
        pl.kernel,
        mesh=mesh,
        out_type=jax.ShapeDtypeStruct((n_total, d), jnp.float32),
        scratch_types=[
            pltpu.VMEM((n_chunks, _CHUNK), jnp.int32),
            pltpu.VMEM((n_chunks, _CHUNK), jnp.int32),
            pltpu.VMEM((b_per_w, d), jnp.float32),
            pltpu.VMEM((b_per_w, d), jnp.float32),
            pltpu.SemaphoreType.DMA,
        ],
    )
    def emb_kernel(tok_hbm, pos_hbm, tokt_hbm, post_hbm, out_hbm,
                   tidx, pidx, trows, prows, sem):
        wid = lax.axis_index("s") * _NC + lax.axis_index("c")
        pltpu.sync_copy(tok_hbm.at[wid], tidx)
        pltpu.sync_copy(pos_hbm.at[wid], pidx)
        copies = []
        for j in range(n_chunks):
            sl = pl.ds(j * _CHUNK, _CHUNK)
            copies.append(pltpu.async_copy(tokt_hbm.at[tidx.at[j]], trows.at[sl], sem))
            copies.append(pltpu.async_copy(post_hbm.at[pidx.at[j]], prows.at[sl], sem))
        for c in copies:
            c.wait()

        def add_row(i, carry):
            for j in range(d // _L):
                sl = pl.ds(j * _L, _L)
                trows[i, sl] = trows[i, sl] + prows[i, sl]
            return carry

        lax.fori_loop(0, b_per_w, add_row, 0)
        pltpu.sync_copy(trows, out_hbm.at[pl.ds(wid * b_per_w, b_per_w)])

    return emb_kernel


def kernel(d_model, max_len, tok_table, pos_table):
    tokens, positions = d_model, max_len
    b, s = tokens.shape
    d = tok_table.shape[1]
    n_total = b * s
    n_chunks = n_total // (_NW * _CHUNK)
    tok = tokens.reshape(_NW, n_chunks, _CHUNK).astype(jnp.int32)
    pos = positions.reshape(_NW, n_chunks, _CHUNK).astype(jnp.int32)
    fn = _build_lookup(n_total, n_chunks, d)
    out = fn(tok, pos, tok_table, pos_table)
    return out.reshape(b, s, d)


# speedup vs baseline: 1.6689x; 1.0111x over previous
import functools

import jax
import jax.numpy as jnp
from jax import lax
from jax.experimental import pallas as pl
from jax.experimental.pallas import tpu as pltpu
from jax.experimental.pallas import tpu_sc as plsc

_INFO = plsc.get_sparse_core_info()
_NC, _NS, _L = _INFO.num_cores, _INFO.num_subcores, _INFO.num_lanes
_NW = _NC * _NS

_CHUNK = 128
_UNROLL = 4


def _build_lookup(b, s, d, n_chunks):
    b_per_w = n_chunks * _CHUNK
    mesh = plsc.VectorSubcoreMesh(core_axis_name="c", subcore_axis_name="s")

    @functools.partial(
        pl.kernel,
        mesh=mesh,
        out_type=jax.ShapeDtypeStruct((b, s, d), jnp.float32),
        scratch_types=[
            pltpu.VMEM((b_per_w,), jnp.int32),
            pltpu.VMEM((b_per_w,), jnp.int32),
            pltpu.VMEM((b_per_w, d), jnp.float32),
            pltpu.VMEM((b_per_w, d), jnp.float32),
            pltpu.SemaphoreType.DMA,
            pltpu.SemaphoreType.DMA,
            pltpu.SemaphoreType.DMA,
        ],
    )
    def emb_kernel(tok_hbm, pos_hbm, tokt_hbm, post_hbm, out_hbm,
                   tidx, pidx, trows, prows, sem0, sem1, osem):
        wid = lax.axis_index("s") * _NC + lax.axis_index("c")
        bb = wid // (s // b_per_w)
        s0 = (wid % (s // b_per_w)) * b_per_w
        pltpu.sync_copy(tok_hbm.at[bb, pl.ds(s0, b_per_w)], tidx)
        pltpu.sync_copy(pos_hbm.at[bb, pl.ds(s0, b_per_w)], pidx)
        sems = [sem0, sem1]
        gathers = []
        for j in range(n_chunks):
            sl = pl.ds(j * _CHUNK, _CHUNK)
            gathers.append((
                pltpu.async_copy(tokt_hbm.at[tidx.at[sl]], trows.at[sl], sems[j]),
                pltpu.async_copy(post_hbm.at[pidx.at[sl]], prows.at[sl], sems[j]),
            ))

        stores = []
        for j in range(n_chunks):
            for g in gathers[j]:
                g.wait()

            def add_rows(i, carry, base=j * _CHUNK):
                for u in range(_UNROLL):
                    r = base + i * _UNROLL + u
                    for k in range(d // _L):
                        sl = pl.ds(k * _L, _L)
                        plsc.addupdate(trows.at[r, sl], prows[r, sl])
                return carry

            lax.fori_loop(0, _CHUNK // _UNROLL, add_rows, 0)
            stores.append(pltpu.async_copy(
                trows.at[pl.ds(j * _CHUNK, _CHUNK)],
                out_hbm.at[bb, pl.ds(s0 + j * _CHUNK, _CHUNK)],
                osem))
        for st in stores:
            st.wait()

    return emb_kernel


def kernel(d_model, max_len, tok_table, pos_table):
    tokens, positions = d_model, max_len
    b, s = tokens.shape
    d = tok_table.shape[1]
    n_chunks = (b * s) // (_NW * _CHUNK)
    fn = _build_lookup(b, s, d, n_chunks)
    return fn(tokens.astype(jnp.int32), positions.astype(jnp.int32),
              tok_table, pos_table)


# speedup vs baseline: 1.6818x; 1.0078x over previous
import functools

import jax
import jax.numpy as jnp
from jax import lax
from jax.experimental import pallas as pl
from jax.experimental.pallas import tpu as pltpu
from jax.experimental.pallas import tpu_sc as plsc

_INFO = plsc.get_sparse_core_info()
_NC, _NS, _L = _INFO.num_cores, _INFO.num_subcores, _INFO.num_lanes
_NW = _NC * _NS

_CHUNK = 64
_UNROLL = 4


def _build_lookup(b, s, d, n_chunks):
    b_per_w = n_chunks * _CHUNK
    mesh = plsc.VectorSubcoreMesh(core_axis_name="c", subcore_axis_name="s")

    @functools.partial(
        pl.kernel,
        mesh=mesh,
        out_type=jax.ShapeDtypeStruct((b, s, d), jnp.float32),
        scratch_types=[
            pltpu.VMEM((b_per_w,), jnp.int32),
            pltpu.VMEM((b_per_w,), jnp.int32),
            pltpu.VMEM((b_per_w, d), jnp.float32),
            pltpu.VMEM((b_per_w, d), jnp.float32),
            pltpu.SemaphoreType.DMA,
        ]
        + [pltpu.SemaphoreType.DMA for _ in range(n_chunks)]
        + [pltpu.SemaphoreType.DMA],
    )
    def emb_kernel(tok_hbm, pos_hbm, tokt_hbm, post_hbm, out_hbm,
                   tidx, pidx, trows, prows, isem, *sems):
        *gsems, osem = sems
        wid = lax.axis_index("s") * _NC + lax.axis_index("c")
        bb = wid // (s // b_per_w)
        s0 = (wid % (s // b_per_w)) * b_per_w
        i0 = pltpu.async_copy(tok_hbm.at[bb, pl.ds(s0, b_per_w)], tidx, isem)
        i1 = pltpu.async_copy(pos_hbm.at[bb, pl.ds(s0, b_per_w)], pidx, isem)
        i0.wait()
        i1.wait()
        gathers = []
        for j in range(n_chunks):
            sl = pl.ds(j * _CHUNK, _CHUNK)
            gathers.append((
                pltpu.async_copy(tokt_hbm.at[tidx.at[sl]], trows.at[sl], gsems[j]),
                pltpu.async_copy(post_hbm.at[pidx.at[sl]], prows.at[sl], gsems[j]),
            ))

        stores = []
        for j in range(n_chunks):
            for g in gathers[j]:
                g.wait()

            @plsc.parallel_loop(j * _CHUNK, (j + 1) * _CHUNK, step=_UNROLL)
            def add_rows(i):
                for u in range(_UNROLL):
                    for k in range(d // _L):
                        sl = pl.ds(k * _L, _L)
                        plsc.addupdate(trows.at[i + u, sl], prows[i + u, sl])

            stores.append(pltpu.async_copy(
                trows.at[pl.ds(j * _CHUNK, _CHUNK)],
                out_hbm.at[bb, pl.ds(s0 + j * _CHUNK, _CHUNK)],
                osem))
        for st in stores:
            st.wait()

    return emb_kernel


def kernel(d_model, max_len, tok_table, pos_table):
    tokens, positions = d_model, max_len
    b, s = tokens.shape
    d = tok_table.shape[1]
    n_chunks = (b * s) // (_NW * _CHUNK)
    fn = _build_lookup(b, s, d, n_chunks)
    return fn(tokens.astype(jnp.int32), positions.astype(jnp.int32),
              tok_table, pos_table)


# speedup vs baseline: 1.7865x; 1.0623x over previous
import functools

import jax
import jax.numpy as jnp
from jax import lax
from jax.experimental import pallas as pl
from jax.experimental.pallas import tpu as pltpu
from jax.experimental.pallas import tpu_sc as plsc

_INFO = plsc.get_sparse_core_info()
_NC, _NS, _L = _INFO.num_cores, _INFO.num_subcores, _INFO.num_lanes
_NW = _NC * _NS

_CHUNK = 64
_UNROLL = 4


def _build_lookup(b, s, d, n_chunks, n_pos):
    b_per_w = n_chunks * _CHUNK
    mesh = plsc.VectorSubcoreMesh(core_axis_name="c", subcore_axis_name="s")

    @functools.partial(
        pl.kernel,
        mesh=mesh,
        out_type=jax.ShapeDtypeStruct((b, s, d), jnp.float32),
        scratch_types=[
            pltpu.VMEM((b_per_w,), jnp.int32),
            pltpu.VMEM((b_per_w,), jnp.int32),
            pltpu.VMEM((b_per_w, d), jnp.float32),
            pltpu.VMEM((b_per_w, d), jnp.float32),
            pltpu.VMEM_SHARED((n_pos, d), jnp.float32),
            pltpu.SemaphoreType.DMA,
            pltpu.SemaphoreType.DMA,
        ]
        + [pltpu.SemaphoreType.DMA for _ in range(n_chunks)]
        + [pltpu.SemaphoreType.DMA],
    )
    def emb_kernel(tok_hbm, pos_hbm, tokt_hbm, post_hbm, out_hbm,
                   tidx, pidx, trows, prows, post_sh, isem, ssem, *sems):
        *gsems, osem = sems
        sid = lax.axis_index("s")
        wid = sid * _NC + lax.axis_index("c")
        bb = wid // (s // b_per_w)
        s0 = (wid % (s // b_per_w)) * b_per_w
        i0 = pltpu.async_copy(tok_hbm.at[bb, pl.ds(s0, b_per_w)], tidx, isem)
        i1 = pltpu.async_copy(pos_hbm.at[bb, pl.ds(s0, b_per_w)], pidx, isem)
        p_per_t = n_pos // _NS
        stg = pltpu.async_copy(
            post_hbm.at[pl.ds(sid * p_per_t, p_per_t)],
            post_sh.at[pl.ds(sid * p_per_t, p_per_t)],
            ssem)
        i0.wait()
        i1.wait()
        tok_gathers = []
        for j in range(n_chunks):
            sl = pl.ds(j * _CHUNK, _CHUNK)
            tok_gathers.append(
                pltpu.async_copy(tokt_hbm.at[tidx.at[sl]], trows.at[sl], gsems[j]))
        stg.wait()
        plsc.subcore_barrier()
        pos_gathers = []
        for j in range(n_chunks):
            sl = pl.ds(j * _CHUNK, _CHUNK)
            pos_gathers.append(
                pltpu.async_copy(post_sh.at[pidx.at[sl]], prows.at[sl], gsems[j]))

        stores = []
        for j in range(n_chunks):
            tok_gathers[j].wait()
            pos_gathers[j].wait()

            @plsc.parallel_loop(j * _CHUNK, (j + 1) * _CHUNK, step=_UNROLL)
            def add_rows(i):
                for u in range(_UNROLL):
                    for k in range(d // _L):
                        sl = pl.ds(k * _L, _L)
                        plsc.addupdate(trows.at[i + u, sl], prows[i + u, sl])

            stores.append(pltpu.async_copy(
                trows.at[pl.ds(j * _CHUNK, _CHUNK)],
                out_hbm.at[bb, pl.ds(s0 + j * _CHUNK, _CHUNK)],
                osem))
        for st in stores:
            st.wait()

    return emb_kernel


def kernel(d_model, max_len, tok_table, pos_table):
    tokens, positions = d_model, max_len
    b, s = tokens.shape
    d = tok_table.shape[1]
    n_chunks = (b * s) // (_NW * _CHUNK)
    fn = _build_lookup(b, s, d, n_chunks, pos_table.shape[0])
    return fn(tokens.astype(jnp.int32), positions.astype(jnp.int32),
              tok_table, pos_table)
